# baseline (speedup 1.0000x reference)
import functools

import jax
import jax.numpy as jnp
from jax import lax
from jax.experimental import pallas as pl
from jax.experimental.pallas import tpu as pltpu
from jax.experimental.pallas import tpu_sc as plsc

TEMPERATURE = 0.05
MAX_SCALE = 1.0 - 0.001
B = 16384
GB = 3 * B
NW = 32
B_PER_W = GB // NW
CHUNK = 128
NCHUNK = B_PER_W // CHUNK


def _sc_gather(xidx, yidx, table_flat):
    mesh = plsc.VectorSubcoreMesh(core_axis_name="c", subcore_axis_name="s")

    @functools.partial(
        pl.kernel,
        mesh=mesh,
        out_type=jax.ShapeDtypeStruct((2 * GB,), jnp.float32),
        scratch_types=[
            pltpu.VMEM((B_PER_W,), jnp.int32),
            pltpu.VMEM((B_PER_W,), jnp.int32),
            pltpu.VMEM((B_PER_W,), jnp.float32),
            pltpu.VMEM((B_PER_W,), jnp.float32),
            pltpu.SemaphoreType.DMA,
        ],
        compiler_params=pltpu.CompilerParams(
            needs_layout_passes=False,
            use_tc_tiling_on_sc=False,
        ),
    )
    def gather_kernel(xidx_hbm, yidx_hbm, table_hbm, out_hbm,
                      xi_v, yi_v, x_v, y_v, sem):
        wid = lax.axis_index("s") * 2 + lax.axis_index("c")
        base = wid * B_PER_W
        pltpu.sync_copy(xidx_hbm.at[pl.ds(base, B_PER_W)], xi_v)
        pltpu.sync_copy(yidx_hbm.at[pl.ds(base, B_PER_W)], yi_v)
        copies = []
        for j in range(NCHUNK):
            sl = pl.ds(j * CHUNK, CHUNK)
            copies.append(pltpu.async_copy(
                table_hbm.at[xi_v.at[sl]], x_v.at[sl], sem))
            copies.append(pltpu.async_copy(
                table_hbm.at[yi_v.at[sl]], y_v.at[sl], sem))
        for c in copies:
            c.wait()
        pltpu.sync_copy(x_v, out_hbm.at[pl.ds(base, B_PER_W)])
        pltpu.sync_copy(y_v, out_hbm.at[pl.ds(GB + base, B_PER_W)])

    return gather_kernel(xidx, yidx, table_flat)


def _loss_body(x_ref, y_ref, s_ref, scale_ref, o_ref):
    s = jnp.clip(scale_ref[0, 0], 0.01, MAX_SCALE)

    def norm(ex, ey):
        n = jnp.sqrt(ex * ex + ey * ey)
        f = s / jnp.maximum(n, 1e-12)
        return ex * f, ey * f

    x1, y1 = norm(x_ref[0], y_ref[0])
    x2, y2 = norm(x_ref[1], y_ref[1])
    x3, y3 = norm(x_ref[2], y_ref[2])

    def hyp_lca(px, py, qx, qy):
        n1 = px * px + py * py
        rx = px / n1
        ry = py / n1
        r2 = rx * rx + ry * ry - 1.0
        ux = qx - rx
        uy = qy - ry
        t = r2 / (ux * ux + uy * uy)
        ax = t * ux + rx
        ay = t * uy + ry
        c = 2.0 * (px * ax + py * ay) / (ax * ax + ay * ay)
        ox = c * ax - px
        oy = c * ay - py
        vx = ox - rx
        vy = oy - ry
        t2 = r2 / (vx * vx + vy * vy)
        wx = t2 * vx + rx
        wy = t2 * vy + ry
        nn = wx * wx + wy * wy
        d = 1.0 + jnp.sqrt(1.0 - nn)
        pn = jnp.sqrt(nn) / d
        return jnp.log((1.0 + pn) / (1.0 - pn))

    inv_t = 1.0 / TEMPERATURE
    l12 = hyp_lca(x1, y1, x2, y2) * inv_t
    l13 = hyp_lca(x1, y1, x3, y3) * inv_t
    l23 = hyp_lca(x2, y2, x3, y3) * inv_t
    m = jnp.maximum(jnp.maximum(l12, l13), l23)
    w1 = jnp.exp(l12 - m)
    w2 = jnp.exp(l13 - m)
    w3 = jnp.exp(l23 - m)
    z = w1 + w2 + w3
    s1 = s_ref[0]
    s2 = s_ref[1]
    s3 = s_ref[2]
    w_ord = (s1 * w1 + s2 * w2 + s3 * w3) / z
    total = (s1 + s2 + s3) - w_ord
    o_ref[0, 0] = jnp.sum(total) * (1.0 / B)


def _tc_loss(xc, yc, sims, scale):
    return pl.pallas_call(
        _loss_body,
        out_shape=jax.ShapeDtypeStruct((1, 1), jnp.float32),
        in_specs=[
            pl.BlockSpec(memory_space=pltpu.VMEM),
            pl.BlockSpec(memory_space=pltpu.VMEM),
            pl.BlockSpec(memory_space=pltpu.VMEM),
            pl.BlockSpec(memory_space=pltpu.SMEM),
        ],
        out_specs=pl.BlockSpec(memory_space=pltpu.SMEM),
    )(xc, yc, sims, scale)


def kernel(triple_ids, similarities, embeddings, scale):
    ids = jnp.transpose(triple_ids.astype(jnp.int32), (1, 0)).reshape(GB)
    xidx = ids * 2
    yidx = xidx + 1
    table_flat = embeddings.astype(jnp.float32).reshape(2 * embeddings.shape[0])
    g = _sc_gather(xidx, yidx, table_flat)
    xc = g[:GB].reshape(3, 128, 128)
    yc = g[GB:].reshape(3, 128, 128)
    sims = jnp.transpose(similarities.astype(jnp.float32), (1, 0)).reshape(3, 128, 128)
    out = _tc_loss(xc, yc, sims, scale.astype(jnp.float32).reshape(1, 1))
    return out[0, 0]


# speedup vs baseline: 17.8131x; 17.8131x over previous
import functools

import jax
import jax.numpy as jnp
from jax import lax
from jax.experimental import pallas as pl
from jax.experimental.pallas import tpu as pltpu
from jax.experimental.pallas import tpu_sc as plsc

TEMPERATURE = 0.05
MAX_SCALE = 1.0 - 0.001
B = 16384
GB = 3 * B
NW = 32
B_PER_W = GB // NW
CHUNK = 128
NCHUNK = B_PER_W // CHUNK


def _sc_gather(ids, xs, ys):
    mesh = plsc.VectorSubcoreMesh(core_axis_name="c", subcore_axis_name="s")

    @functools.partial(
        pl.kernel,
        mesh=mesh,
        out_type=jax.ShapeDtypeStruct((2 * GB,), jnp.float32),
        scratch_types=[
            pltpu.VMEM((B_PER_W,), jnp.int32),
            pltpu.VMEM((B_PER_W,), jnp.float32),
            pltpu.VMEM((B_PER_W,), jnp.float32),
            pltpu.SemaphoreType.DMA,
        ],
        compiler_params=pltpu.CompilerParams(
            needs_layout_passes=False,
            use_tc_tiling_on_sc=False,
        ),
    )
    def gather_kernel(ids_hbm, xs_hbm, ys_hbm, out_hbm, i_v, x_v, y_v, sem):
        wid = lax.axis_index("s") * 2 + lax.axis_index("c")
        base = wid * B_PER_W
        pltpu.sync_copy(ids_hbm.at[pl.ds(base, B_PER_W)], i_v)
        copies = []
        for j in range(NCHUNK):
            sl = pl.ds(j * CHUNK, CHUNK)
            copies.append(pltpu.async_copy(
                xs_hbm.at[i_v.at[sl]], x_v.at[sl], sem))
            copies.append(pltpu.async_copy(
                ys_hbm.at[i_v.at[sl]], y_v.at[sl], sem))
        for c in copies:
            c.wait()
        pltpu.sync_copy(x_v, out_hbm.at[pl.ds(base, B_PER_W)])
        pltpu.sync_copy(y_v, out_hbm.at[pl.ds(GB + base, B_PER_W)])

    return gather_kernel(ids, xs, ys)


def _loss_body(x_ref, y_ref, s_ref, scale_ref, o_ref):
    s = jnp.clip(scale_ref[0, 0], 0.01, MAX_SCALE)

    def norm(ex, ey):
        n = jnp.sqrt(ex * ex + ey * ey)
        f = s / jnp.maximum(n, 1e-12)
        return ex * f, ey * f

    x1, y1 = norm(x_ref[0], y_ref[0])
    x2, y2 = norm(x_ref[1], y_ref[1])
    x3, y3 = norm(x_ref[2], y_ref[2])

    def hyp_lca(px, py, qx, qy):
        n1 = px * px + py * py
        rx = px / n1
        ry = py / n1
        r2 = rx * rx + ry * ry - 1.0
        ux = qx - rx
        uy = qy - ry
        t = r2 / (ux * ux + uy * uy)
        ax = t * ux + rx
        ay = t * uy + ry
        c = 2.0 * (px * ax + py * ay) / (ax * ax + ay * ay)
        ox = c * ax - px
        oy = c * ay - py
        vx = ox - rx
        vy = oy - ry
        t2 = r2 / (vx * vx + vy * vy)
        wx = t2 * vx + rx
        wy = t2 * vy + ry
        nn = wx * wx + wy * wy
        d = 1.0 + jnp.sqrt(1.0 - nn)
        pn = jnp.sqrt(nn) / d
        return jnp.log((1.0 + pn) / (1.0 - pn))

    inv_t = 1.0 / TEMPERATURE
    l12 = hyp_lca(x1, y1, x2, y2) * inv_t
    l13 = hyp_lca(x1, y1, x3, y3) * inv_t
    l23 = hyp_lca(x2, y2, x3, y3) * inv_t
    m = jnp.maximum(jnp.maximum(l12, l13), l23)
    w1 = jnp.exp(l12 - m)
    w2 = jnp.exp(l13 - m)
    w3 = jnp.exp(l23 - m)
    z = w1 + w2 + w3
    s1 = s_ref[0]
    s2 = s_ref[1]
    s3 = s_ref[2]
    w_ord = (s1 * w1 + s2 * w2 + s3 * w3) / z
    total = (s1 + s2 + s3) - w_ord
    o_ref[0, 0] = jnp.sum(total) * (1.0 / B)


def _tc_loss(xc, yc, sims, scale):
    return pl.pallas_call(
        _loss_body,
        out_shape=jax.ShapeDtypeStruct((1, 1), jnp.float32),
        in_specs=[
            pl.BlockSpec(memory_space=pltpu.VMEM),
            pl.BlockSpec(memory_space=pltpu.VMEM),
            pl.BlockSpec(memory_space=pltpu.VMEM),
            pl.BlockSpec(memory_space=pltpu.SMEM),
        ],
        out_specs=pl.BlockSpec(memory_space=pltpu.SMEM),
    )(xc, yc, sims, scale)


def kernel(triple_ids, similarities, embeddings, scale):
    ids = jnp.transpose(triple_ids.astype(jnp.int32), (1, 0)).reshape(GB)
    n = embeddings.shape[0]
    xs = jax.lax.slice(embeddings, (0, 0), (n, 1)).reshape(n)
    ys = jax.lax.slice(embeddings, (0, 1), (n, 2)).reshape(n)
    g = _sc_gather(ids, xs, ys)
    xc = g[:GB].reshape(3, 128, 128)
    yc = g[GB:].reshape(3, 128, 128)
    sims = jnp.transpose(similarities.astype(jnp.float32), (1, 0)).reshape(3, 128, 128)
    out = _tc_loss(xc, yc, sims, scale.astype(jnp.float32).reshape(1, 1))
    return out[0, 0]


# speedup vs baseline: 34.3295x; 1.9272x over previous
import functools

import jax
import jax.numpy as jnp
from jax import lax
from jax.experimental import pallas as pl
from jax.experimental.pallas import tpu as pltpu
from jax.experimental.pallas import tpu_sc as plsc

TEMPERATURE = 0.05
MAX_SCALE = 1.0 - 0.001
B = 16384
GB = 3 * B
NW = 32
B_PER_W = GB // NW
CHUNK = 128
NCHUNK = B_PER_W // CHUNK


def _sc_gather(xidx, yidx, table):
    mesh = plsc.VectorSubcoreMesh(core_axis_name="c", subcore_axis_name="s")

    @functools.partial(
        pl.kernel,
        mesh=mesh,
        out_type=[
            jax.ShapeDtypeStruct((GB,), jnp.float32),
            jax.ShapeDtypeStruct((GB,), jnp.float32),
        ],
        scratch_types=[
            pltpu.VMEM((B_PER_W,), jnp.int32),
            pltpu.VMEM((B_PER_W,), jnp.int32),
            pltpu.VMEM((B_PER_W,), jnp.float32),
            pltpu.VMEM((B_PER_W,), jnp.float32),
            pltpu.SemaphoreType.DMA,
        ],
        compiler_params=pltpu.CompilerParams(
            needs_layout_passes=False,
            use_tc_tiling_on_sc=False,
        ),
    )
    def gather_kernel(xidx_hbm, yidx_hbm, table_hbm, outx_hbm, outy_hbm,
                      xi_v, yi_v, x_v, y_v, sem):
        wid = lax.axis_index("s") * 2 + lax.axis_index("c")
        base = wid * B_PER_W
        pltpu.sync_copy(xidx_hbm.at[pl.ds(base, B_PER_W)], xi_v)
        pltpu.sync_copy(yidx_hbm.at[pl.ds(base, B_PER_W)], yi_v)
        copies = []
        for j in range(NCHUNK):
            sl = pl.ds(j * CHUNK, CHUNK)
            copies.append(pltpu.async_copy(
                table_hbm.at[xi_v.at[sl]], x_v.at[sl], sem))
            copies.append(pltpu.async_copy(
                table_hbm.at[yi_v.at[sl]], y_v.at[sl], sem))
        for c in copies:
            c.wait()
        pltpu.sync_copy(x_v, outx_hbm.at[pl.ds(base, B_PER_W)])
        pltpu.sync_copy(y_v, outy_hbm.at[pl.ds(base, B_PER_W)])

    return gather_kernel(xidx, yidx, table)


def _loss_body(x_ref, y_ref, s_ref, scale_ref, o_ref):
    s = jnp.clip(scale_ref[0, 0], 0.01, MAX_SCALE)

    def norm(ex, ey):
        n = jnp.sqrt(ex * ex + ey * ey)
        f = s / jnp.maximum(n, 1e-12)
        return ex * f, ey * f

    x1, y1 = norm(x_ref[0], y_ref[0])
    x2, y2 = norm(x_ref[1], y_ref[1])
    x3, y3 = norm(x_ref[2], y_ref[2])

    def hyp_lca(px, py, qx, qy):
        n1 = px * px + py * py
        rx = px / n1
        ry = py / n1
        r2 = rx * rx + ry * ry - 1.0
        ux = qx - rx
        uy = qy - ry
        t = r2 / (ux * ux + uy * uy)
        ax = t * ux + rx
        ay = t * uy + ry
        c = 2.0 * (px * ax + py * ay) / (ax * ax + ay * ay)
        ox = c * ax - px
        oy = c * ay - py
        vx = ox - rx
        vy = oy - ry
        t2 = r2 / (vx * vx + vy * vy)
        wx = t2 * vx + rx
        wy = t2 * vy + ry
        nn = wx * wx + wy * wy
        d = 1.0 + jnp.sqrt(1.0 - nn)
        pn = jnp.sqrt(nn) / d
        return jnp.log((1.0 + pn) / (1.0 - pn))

    inv_t = 1.0 / TEMPERATURE
    l12 = hyp_lca(x1, y1, x2, y2) * inv_t
    l13 = hyp_lca(x1, y1, x3, y3) * inv_t
    l23 = hyp_lca(x2, y2, x3, y3) * inv_t
    m = jnp.maximum(jnp.maximum(l12, l13), l23)
    w1 = jnp.exp(l12 - m)
    w2 = jnp.exp(l13 - m)
    w3 = jnp.exp(l23 - m)
    z = w1 + w2 + w3
    s1 = s_ref[0]
    s2 = s_ref[1]
    s3 = s_ref[2]
    w_ord = (s1 * w1 + s2 * w2 + s3 * w3) / z
    total = (s1 + s2 + s3) - w_ord
    o_ref[0, 0] = jnp.sum(total) * (1.0 / B)


def _tc_loss(xc, yc, sims, scale):
    return pl.pallas_call(
        _loss_body,
        out_shape=jax.ShapeDtypeStruct((1, 1), jnp.float32),
        in_specs=[
            pl.BlockSpec(memory_space=pltpu.VMEM),
            pl.BlockSpec(memory_space=pltpu.VMEM),
            pl.BlockSpec(memory_space=pltpu.VMEM),
            pl.BlockSpec(memory_space=pltpu.SMEM),
        ],
        out_specs=pl.BlockSpec(memory_space=pltpu.SMEM),
    )(xc, yc, sims, scale)


def kernel(triple_ids, similarities, embeddings, scale):
    ids = jnp.transpose(triple_ids.astype(jnp.int32), (1, 0)).reshape(GB)
    n = embeddings.shape[0]
    table = jnp.transpose(embeddings.astype(jnp.float32)).reshape(2 * n)
    gx, gy = _sc_gather(ids, ids + n, table)
    xc = gx.reshape(3, 128, 128)
    yc = gy.reshape(3, 128, 128)
    sims = jnp.transpose(similarities.astype(jnp.float32), (1, 0)).reshape(3, 128, 128)
    out = _tc_loss(xc, yc, sims, scale.astype(jnp.float32).reshape(1, 1))
    return out[0, 0]
